# baseline (device time: 60763 ns/iter reference)
import jax
import jax.numpy as jnp
from jax import lax
from jax.experimental import pallas as pl
from jax.experimental.pallas import tpu as pltpu

N_DEV = 4
N_SUB = 2

CW, CCW = 0, 1


def _fused(lidx, packed, ccounts, mask, table):
    n = lidx.shape[0]
    v_per, d = table.shape
    half = n // 2
    s = half // N_DEV
    s2 = s // N_SUB

    def body(lidx_ref, packed_ref, ccount_ref, mask_ref, table_ref, out_ref,
             gath_ref, stage_cw, stage_ccw, send_sems, recv_sems, ag_recv,
             gsem):
        my = lax.axis_index("i")
        left = lax.rem(my + N_DEV - 1, N_DEV)
        right = lax.rem(my + 1, N_DEV)

        def rmod(v):
            return lax.rem(v + 2 * N_DEV, N_DEV)

        def row_dma(j, sem):
            return pltpu.make_async_copy(
                table_ref.at[pl.ds(lidx_ref[j], 1), :],
                gath_ref.at[pl.ds(j, 1), :],
                sem,
            )

        def stage_chunks(k):
            return rmod(my - k), N_DEV + rmod(my + k)

        def issue_stage(k):
            for c8 in stage_chunks(k):
                base = c8 * s

                def f(i, c):
                    row_dma(packed_ref[base + i], gsem.at[k]).start()
                    return c
                lax.fori_loop(0, ccount_ref[c8], f, 0)

        def finish_stage(k):
            cr, cl = stage_chunks(k)
            for c8 in (cr, cl):
                def f(i, c):
                    row_dma(0, gsem.at[k]).wait()
                    return c
                lax.fori_loop(0, ccount_ref[c8], f, 0)
            for c8 in (cr, cl):
                off = c8 * s
                out_ref[pl.ds(off, s), :] = (
                    gath_ref[pl.ds(off, s), :] * mask_ref[pl.ds(off, s), :]
                ).astype(out_ref.dtype)

        def rs_chunk(dirn, h):
            if dirn == CW:
                return rmod(my - h)
            return N_DEV + rmod(my + h)

        def rs_acc_chunk(dirn, h):
            if dirn == CW:
                return rmod(my - h - 1)
            return N_DEV + rmod(my + h + 1)

        def ag_chunk(dirn, h):
            if dirn == CW:
                return rmod(my + 1 - h)
            return N_DEV + rmod(my - 1 + h)

        def peer(dirn):
            return right if dirn == CW else left

        def sub(ref, c8, k):
            return ref.at[pl.ds(c8 * s + k * s2, s2), :]

        def rs_desc(dirn, h, k):
            stage = stage_cw if dirn == CW else stage_ccw
            return pltpu.make_async_remote_copy(
                src_ref=sub(out_ref, rs_chunk(dirn, h), k),
                dst_ref=stage.at[h, pl.ds(k * s2, s2), :],
                send_sem=send_sems.at[dirn, h, k],
                recv_sem=recv_sems.at[dirn, h, k],
                device_id=(peer(dirn),),
                device_id_type=pl.DeviceIdType.MESH,
            )

        def ag_desc(dirn, h, k):
            region = sub(out_ref, ag_chunk(dirn, h), k)
            return pltpu.make_async_remote_copy(
                src_ref=region,
                dst_ref=region,
                send_sem=send_sems.at[dirn, h, k],
                recv_sem=ag_recv.at[dirn, h, k],
                device_id=(peer(dirn),),
                device_id_type=pl.DeviceIdType.MESH,
            )

        def rs_add(dirn, h, k):
            stage = stage_cw if dirn == CW else stage_ccw
            c8 = rs_acc_chunk(dirn, h)
            out_ref[pl.ds(c8 * s + k * s2, s2), :] += (
                stage[h, pl.ds(k * s2, s2), :]
            )

        for k in range(N_DEV):
            issue_stage(k)

        barrier_sem = pltpu.get_barrier_semaphore()
        for nbr in (left, right):
            pl.semaphore_signal(
                barrier_sem, inc=1,
                device_id=(nbr,), device_id_type=pl.DeviceIdType.MESH,
            )

        finish_stage(0)
        pl.semaphore_wait(barrier_sem, 2)

        ag_descs = {}
        for dirn in (CW, CCW):
            for k in range(N_SUB):
                rs_desc(dirn, 0, k).start()

        for h in range(N_DEV - 1):
            finish_stage(h + 1)
            for k in range(N_SUB):
                for dirn in (CW, CCW):
                    rs_desc(dirn, h, k).wait()
                    rs_add(dirn, h, k)
                    if h + 1 < N_DEV - 1:
                        rs_desc(dirn, h + 1, k).start()
                    else:
                        ag = ag_desc(dirn, 0, k)
                        ag_descs[dirn, 0, k] = ag
                        ag.start()

        for h in range(1, N_DEV - 1):
            for k in range(N_SUB):
                for dirn in (CW, CCW):
                    ag_descs[dirn, h - 1, k].wait_recv()
                    ag = ag_desc(dirn, h, k)
                    ag_descs[dirn, h, k] = ag
                    ag.start()
        for k in range(N_SUB):
            for dirn in (CW, CCW):
                ag_descs[dirn, N_DEV - 2, k].wait_recv()
        for (dirn, h, k), ag in ag_descs.items():
            ag.wait_send()

    return pl.pallas_call(
        body,
        out_shape=jax.ShapeDtypeStruct((n, d), jnp.bfloat16),
        in_specs=[
            pl.BlockSpec(memory_space=pltpu.SMEM),
            pl.BlockSpec(memory_space=pltpu.SMEM),
            pl.BlockSpec(memory_space=pltpu.SMEM),
            pl.BlockSpec(memory_space=pltpu.VMEM),
            pl.BlockSpec(memory_space=pltpu.HBM),
        ],
        out_specs=pl.BlockSpec(memory_space=pltpu.VMEM),
        scratch_shapes=[
            pltpu.VMEM((n, d), jnp.float32),
            pltpu.VMEM((N_DEV - 1, s, d), jnp.bfloat16),
            pltpu.VMEM((N_DEV - 1, s, d), jnp.bfloat16),
            pltpu.SemaphoreType.DMA((2, N_DEV - 1, N_SUB)),
            pltpu.SemaphoreType.DMA((2, N_DEV - 1, N_SUB)),
            pltpu.SemaphoreType.DMA((2, N_DEV - 1, N_SUB)),
            pltpu.SemaphoreType.DMA((N_DEV,)),
        ],
        compiler_params=pltpu.CompilerParams(collective_id=0),
    )(lidx, packed, ccounts, mask, table)


def kernel(table, idx):
    v_per, _ = table.shape
    n = idx.shape[0]
    my = lax.axis_index("i")
    lidx = idx.astype(jnp.int32) - my * v_per
    owned = ((lidx >= 0) & (lidx < v_per)).astype(jnp.int32)
    mask = owned.astype(jnp.float32)[:, None]
    s = n // (2 * N_DEV)
    owned2d = owned.reshape(2 * N_DEV, s)
    ccounts = owned2d.sum(axis=1, dtype=jnp.int32)
    order = jnp.argsort(1 - owned2d, axis=1, stable=True)
    packed = (
        jnp.arange(2 * N_DEV, dtype=jnp.int32)[:, None] * s
        + order.astype(jnp.int32)
    ).reshape(-1)
    return _fused(jnp.clip(lidx, 0, v_per - 1), packed, ccounts, mask, table)


# device time: 51408 ns/iter; 1.1820x vs baseline; 1.1820x over previous
import jax
import jax.numpy as jnp
from jax import lax
from jax.experimental import pallas as pl
from jax.experimental.pallas import tpu as pltpu

N_DEV = 4
N_SUB = 2

CW, CCW = 0, 1


def _fused(lidx, packed, ccounts, mask, table):
    n = lidx.shape[0]
    v_per, d = table.shape
    half = n // 2
    s = half // N_DEV
    s2 = s // N_SUB

    def body(lidx_ref, packed_ref, ccount_ref, mask_ref, table_ref, out_ref,
             gath_ref, stage_cw, stage_ccw, send_sems, recv_sems, ag_recv,
             gsem):
        my = lax.axis_index("i")
        left = lax.rem(my + N_DEV - 1, N_DEV)
        right = lax.rem(my + 1, N_DEV)

        def rmod(v):
            return lax.rem(v + 2 * N_DEV, N_DEV)

        def row_dma(j, sem):
            return pltpu.make_async_copy(
                table_ref.at[pl.ds(lidx_ref[j], 1), :],
                gath_ref.at[pl.ds(j, 1), :],
                sem,
            )

        def stage_chunks(k):
            return rmod(my - k), N_DEV + rmod(my + k)

        def issue_stage(k):
            for c8 in stage_chunks(k):
                base = c8 * s

                def f(i, c):
                    row_dma(packed_ref[base + i], gsem.at[k]).start()
                    return c
                lax.fori_loop(0, ccount_ref[c8], f, 0)

        def finish_stage(k):
            cr, cl = stage_chunks(k)
            for c8 in (cr, cl):
                def f(i, c):
                    row_dma(0, gsem.at[k]).wait()
                    return c
                lax.fori_loop(0, ccount_ref[c8], f, 0)
            for c8 in (cr, cl):
                off = c8 * s
                out_ref[pl.ds(off, s), :] = (
                    gath_ref[pl.ds(off, s), :] * mask_ref[pl.ds(off, s), :]
                ).astype(out_ref.dtype)

        def rs_chunk(dirn, h):
            if dirn == CW:
                return rmod(my - h)
            return N_DEV + rmod(my + h)

        def rs_acc_chunk(dirn, h):
            if dirn == CW:
                return rmod(my - h - 1)
            return N_DEV + rmod(my + h + 1)

        def ag_chunk(dirn, h):
            if dirn == CW:
                return rmod(my + 1 - h)
            return N_DEV + rmod(my - 1 + h)

        def peer(dirn):
            return right if dirn == CW else left

        def sub(ref, c8, k):
            return ref.at[pl.ds(c8 * s + k * s2, s2), :]

        def rs_desc(dirn, h, k):
            stage = stage_cw if dirn == CW else stage_ccw
            return pltpu.make_async_remote_copy(
                src_ref=sub(out_ref, rs_chunk(dirn, h), k),
                dst_ref=stage.at[h, pl.ds(k * s2, s2), :],
                send_sem=send_sems.at[dirn, h, k],
                recv_sem=recv_sems.at[dirn, h, k],
                device_id=(peer(dirn),),
                device_id_type=pl.DeviceIdType.MESH,
            )

        def ag_desc(dirn, h, k):
            region = sub(out_ref, ag_chunk(dirn, h), k)
            return pltpu.make_async_remote_copy(
                src_ref=region,
                dst_ref=region,
                send_sem=send_sems.at[dirn, h, k],
                recv_sem=ag_recv.at[dirn, h, k],
                device_id=(peer(dirn),),
                device_id_type=pl.DeviceIdType.MESH,
            )

        def rs_add(dirn, h, k):
            stage = stage_cw if dirn == CW else stage_ccw
            c8 = rs_acc_chunk(dirn, h)
            out_ref[pl.ds(c8 * s + k * s2, s2), :] += (
                stage[h, pl.ds(k * s2, s2), :]
            )

        issue_stage(0)

        barrier_sem = pltpu.get_barrier_semaphore()
        for nbr in (left, right):
            pl.semaphore_signal(
                barrier_sem, inc=1,
                device_id=(nbr,), device_id_type=pl.DeviceIdType.MESH,
            )

        finish_stage(0)
        pl.semaphore_wait(barrier_sem, 2)

        ag_descs = {}
        for dirn in (CW, CCW):
            for k in range(N_SUB):
                rs_desc(dirn, 0, k).start()

        for h in range(N_DEV - 1):
            issue_stage(h + 1)
            finish_stage(h + 1)
            for k in range(N_SUB):
                for dirn in (CW, CCW):
                    rs_desc(dirn, h, k).wait()
                    rs_add(dirn, h, k)
                    if h + 1 < N_DEV - 1:
                        rs_desc(dirn, h + 1, k).start()
                    else:
                        ag = ag_desc(dirn, 0, k)
                        ag_descs[dirn, 0, k] = ag
                        ag.start()

        for h in range(1, N_DEV - 1):
            for k in range(N_SUB):
                for dirn in (CW, CCW):
                    ag_descs[dirn, h - 1, k].wait_recv()
                    ag = ag_desc(dirn, h, k)
                    ag_descs[dirn, h, k] = ag
                    ag.start()
        for k in range(N_SUB):
            for dirn in (CW, CCW):
                ag_descs[dirn, N_DEV - 2, k].wait_recv()
        for (dirn, h, k), ag in ag_descs.items():
            ag.wait_send()

    return pl.pallas_call(
        body,
        out_shape=jax.ShapeDtypeStruct((n, d), jnp.bfloat16),
        in_specs=[
            pl.BlockSpec(memory_space=pltpu.SMEM),
            pl.BlockSpec(memory_space=pltpu.SMEM),
            pl.BlockSpec(memory_space=pltpu.SMEM),
            pl.BlockSpec(memory_space=pltpu.VMEM),
            pl.BlockSpec(memory_space=pltpu.HBM),
        ],
        out_specs=pl.BlockSpec(memory_space=pltpu.VMEM),
        scratch_shapes=[
            pltpu.VMEM((n, d), jnp.float32),
            pltpu.VMEM((N_DEV - 1, s, d), jnp.bfloat16),
            pltpu.VMEM((N_DEV - 1, s, d), jnp.bfloat16),
            pltpu.SemaphoreType.DMA((2, N_DEV - 1, N_SUB)),
            pltpu.SemaphoreType.DMA((2, N_DEV - 1, N_SUB)),
            pltpu.SemaphoreType.DMA((2, N_DEV - 1, N_SUB)),
            pltpu.SemaphoreType.DMA((N_DEV,)),
        ],
        compiler_params=pltpu.CompilerParams(collective_id=0),
    )(lidx, packed, ccounts, mask, table)


def kernel(table, idx):
    v_per, _ = table.shape
    n = idx.shape[0]
    my = lax.axis_index("i")
    lidx = idx.astype(jnp.int32) - my * v_per
    owned = ((lidx >= 0) & (lidx < v_per)).astype(jnp.int32)
    mask = owned.astype(jnp.float32)[:, None]
    s = n // (2 * N_DEV)
    owned2d = owned.reshape(2 * N_DEV, s)
    ccounts = owned2d.sum(axis=1, dtype=jnp.int32)
    order = jnp.argsort(1 - owned2d, axis=1, stable=True)
    packed = (
        jnp.arange(2 * N_DEV, dtype=jnp.int32)[:, None] * s
        + order.astype(jnp.int32)
    ).reshape(-1)
    return _fused(jnp.clip(lidx, 0, v_per - 1), packed, ccounts, mask, table)


# device time: 50812 ns/iter; 1.1958x vs baseline; 1.0117x over previous
import jax
import jax.numpy as jnp
from jax import lax
from jax.experimental import pallas as pl
from jax.experimental.pallas import tpu as pltpu

N_DEV = 4
N_SUB = 4

CW, CCW = 0, 1


def _fused(lidx, packed, ccounts, mask, table):
    n = lidx.shape[0]
    v_per, d = table.shape
    half = n // 2
    s = half // N_DEV
    s2 = s // N_SUB

    def body(lidx_ref, packed_ref, ccount_ref, mask_ref, table_ref, out_ref,
             gath_ref, stage_cw, stage_ccw, send_sems, recv_sems, ag_recv,
             gsem):
        my = lax.axis_index("i")
        left = lax.rem(my + N_DEV - 1, N_DEV)
        right = lax.rem(my + 1, N_DEV)

        def rmod(v):
            return lax.rem(v + 2 * N_DEV, N_DEV)

        def row_dma(j, sem):
            return pltpu.make_async_copy(
                table_ref.at[pl.ds(lidx_ref[j], 1), :],
                gath_ref.at[pl.ds(j, 1), :],
                sem,
            )

        def stage_chunks(k):
            return rmod(my - k), N_DEV + rmod(my + k)

        def issue_stage(k):
            for c8 in stage_chunks(k):
                base = c8 * s

                def f(i, c):
                    row_dma(packed_ref[base + i], gsem.at[k]).start()
                    return c
                lax.fori_loop(0, ccount_ref[c8], f, 0)

        def finish_stage(k):
            cr, cl = stage_chunks(k)
            for c8 in (cr, cl):
                def f(i, c):
                    row_dma(0, gsem.at[k]).wait()
                    return c
                lax.fori_loop(0, ccount_ref[c8], f, 0)
            for c8 in (cr, cl):
                off = c8 * s
                out_ref[pl.ds(off, s), :] = (
                    gath_ref[pl.ds(off, s), :] * mask_ref[pl.ds(off, s), :]
                ).astype(out_ref.dtype)

        def rs_chunk(dirn, h):
            if dirn == CW:
                return rmod(my - h)
            return N_DEV + rmod(my + h)

        def rs_acc_chunk(dirn, h):
            if dirn == CW:
                return rmod(my - h - 1)
            return N_DEV + rmod(my + h + 1)

        def ag_chunk(dirn, h):
            if dirn == CW:
                return rmod(my + 1 - h)
            return N_DEV + rmod(my - 1 + h)

        def peer(dirn):
            return right if dirn == CW else left

        def sub(ref, c8, k):
            return ref.at[pl.ds(c8 * s + k * s2, s2), :]

        def rs_desc(dirn, h, k):
            stage = stage_cw if dirn == CW else stage_ccw
            return pltpu.make_async_remote_copy(
                src_ref=sub(out_ref, rs_chunk(dirn, h), k),
                dst_ref=stage.at[h, pl.ds(k * s2, s2), :],
                send_sem=send_sems.at[dirn, h, k],
                recv_sem=recv_sems.at[dirn, h, k],
                device_id=(peer(dirn),),
                device_id_type=pl.DeviceIdType.MESH,
            )

        def ag_desc(dirn, h, k):
            region = sub(out_ref, ag_chunk(dirn, h), k)
            return pltpu.make_async_remote_copy(
                src_ref=region,
                dst_ref=region,
                send_sem=send_sems.at[dirn, h, k],
                recv_sem=ag_recv.at[dirn, h, k],
                device_id=(peer(dirn),),
                device_id_type=pl.DeviceIdType.MESH,
            )

        def rs_add(dirn, h, k):
            stage = stage_cw if dirn == CW else stage_ccw
            c8 = rs_acc_chunk(dirn, h)
            out_ref[pl.ds(c8 * s + k * s2, s2), :] += (
                stage[h, pl.ds(k * s2, s2), :]
            )

        issue_stage(0)

        barrier_sem = pltpu.get_barrier_semaphore()
        for nbr in (left, right):
            pl.semaphore_signal(
                barrier_sem, inc=1,
                device_id=(nbr,), device_id_type=pl.DeviceIdType.MESH,
            )

        finish_stage(0)
        pl.semaphore_wait(barrier_sem, 2)

        ag_descs = {}
        for dirn in (CW, CCW):
            for k in range(N_SUB):
                rs_desc(dirn, 0, k).start()

        for h in range(N_DEV - 1):
            issue_stage(h + 1)
            finish_stage(h + 1)
            for k in range(N_SUB):
                for dirn in (CW, CCW):
                    rs_desc(dirn, h, k).wait()
                    rs_add(dirn, h, k)
                    if h + 1 < N_DEV - 1:
                        rs_desc(dirn, h + 1, k).start()
                    else:
                        ag = ag_desc(dirn, 0, k)
                        ag_descs[dirn, 0, k] = ag
                        ag.start()

        for h in range(1, N_DEV - 1):
            for k in range(N_SUB):
                for dirn in (CW, CCW):
                    ag_descs[dirn, h - 1, k].wait_recv()
                    ag = ag_desc(dirn, h, k)
                    ag_descs[dirn, h, k] = ag
                    ag.start()
        for k in range(N_SUB):
            for dirn in (CW, CCW):
                ag_descs[dirn, N_DEV - 2, k].wait_recv()
        for (dirn, h, k), ag in ag_descs.items():
            ag.wait_send()

    return pl.pallas_call(
        body,
        out_shape=jax.ShapeDtypeStruct((n, d), jnp.bfloat16),
        in_specs=[
            pl.BlockSpec(memory_space=pltpu.SMEM),
            pl.BlockSpec(memory_space=pltpu.SMEM),
            pl.BlockSpec(memory_space=pltpu.SMEM),
            pl.BlockSpec(memory_space=pltpu.VMEM),
            pl.BlockSpec(memory_space=pltpu.HBM),
        ],
        out_specs=pl.BlockSpec(memory_space=pltpu.VMEM),
        scratch_shapes=[
            pltpu.VMEM((n, d), jnp.float32),
            pltpu.VMEM((N_DEV - 1, s, d), jnp.bfloat16),
            pltpu.VMEM((N_DEV - 1, s, d), jnp.bfloat16),
            pltpu.SemaphoreType.DMA((2, N_DEV - 1, N_SUB)),
            pltpu.SemaphoreType.DMA((2, N_DEV - 1, N_SUB)),
            pltpu.SemaphoreType.DMA((2, N_DEV - 1, N_SUB)),
            pltpu.SemaphoreType.DMA((N_DEV,)),
        ],
        compiler_params=pltpu.CompilerParams(collective_id=0),
    )(lidx, packed, ccounts, mask, table)


def kernel(table, idx):
    v_per, _ = table.shape
    n = idx.shape[0]
    my = lax.axis_index("i")
    lidx = idx.astype(jnp.int32) - my * v_per
    owned = ((lidx >= 0) & (lidx < v_per)).astype(jnp.int32)
    mask = owned.astype(jnp.float32)[:, None]
    s = n // (2 * N_DEV)
    owned2d = owned.reshape(2 * N_DEV, s)
    ccounts = owned2d.sum(axis=1, dtype=jnp.int32)
    order = jnp.argsort(1 - owned2d, axis=1, stable=True)
    packed = (
        jnp.arange(2 * N_DEV, dtype=jnp.int32)[:, None] * s
        + order.astype(jnp.int32)
    ).reshape(-1)
    return _fused(jnp.clip(lidx, 0, v_per - 1), packed, ccounts, mask, table)


# device time: 48775 ns/iter; 1.2458x vs baseline; 1.0418x over previous
import jax
import jax.numpy as jnp
from jax import lax
from jax.experimental import pallas as pl
from jax.experimental.pallas import tpu as pltpu

N_DEV = 4
N_SUB = 4

CW, CCW = 0, 1


def _fused(lidx, packed, ccounts, mask, table):
    n = lidx.shape[0]
    v_per, d = table.shape
    half = n // 2
    s = half // N_DEV
    s2 = s // N_SUB

    def body(lidx_ref, packed_ref, ccount_ref, mask_ref, table_ref, out_ref,
             gath_ref, stage_cw, stage_ccw, send_sems, recv_sems, ag_recv,
             gsem):
        my = lax.axis_index("i")
        left = lax.rem(my + N_DEV - 1, N_DEV)
        right = lax.rem(my + 1, N_DEV)

        def rmod(v):
            return lax.rem(v + 2 * N_DEV, N_DEV)

        def row_dma(j, sem):
            return pltpu.make_async_copy(
                table_ref.at[pl.ds(lidx_ref[j], 1), :],
                gath_ref.at[pl.ds(j, 1), :],
                sem,
            )

        def issue_group(g, sem):
            base = g * s2

            def f(i, c):
                row_dma(packed_ref[base + i], sem).start()
                return c
            lax.fori_loop(0, ccount_ref[g], f, 0)

        def drain_group(g, sem):
            def f(i, c):
                row_dma(0, sem).wait()
                return c
            lax.fori_loop(0, ccount_ref[g], f, 0)

        def convert(off, rows):
            out_ref[pl.ds(off, rows), :] = (
                gath_ref[pl.ds(off, rows), :] * mask_ref[pl.ds(off, rows), :]
            ).astype(out_ref.dtype)

        def stage_chunks(k):
            return rmod(my - k), N_DEV + rmod(my + k)

        def issue_stage(k):
            for c8 in stage_chunks(k):
                for k2 in range(N_SUB):
                    sem = gsem.at[N_DEV - 1 + k2] if k == 0 else gsem.at[k - 1]
                    issue_group(c8 * N_SUB + k2, sem)

        def finish_stage(k):
            assert k > 0
            cr, cl = stage_chunks(k)
            for c8 in (cr, cl):
                for k2 in range(N_SUB):
                    drain_group(c8 * N_SUB + k2, gsem.at[k - 1])
            for c8 in (cr, cl):
                convert(c8 * s, s)

        def rs_chunk(dirn, h):
            if dirn == CW:
                return rmod(my - h)
            return N_DEV + rmod(my + h)

        def rs_acc_chunk(dirn, h):
            if dirn == CW:
                return rmod(my - h - 1)
            return N_DEV + rmod(my + h + 1)

        def ag_chunk(dirn, h):
            if dirn == CW:
                return rmod(my + 1 - h)
            return N_DEV + rmod(my - 1 + h)

        def peer(dirn):
            return right if dirn == CW else left

        def sub(ref, c8, k):
            return ref.at[pl.ds(c8 * s + k * s2, s2), :]

        def rs_desc(dirn, h, k):
            stage = stage_cw if dirn == CW else stage_ccw
            return pltpu.make_async_remote_copy(
                src_ref=sub(out_ref, rs_chunk(dirn, h), k),
                dst_ref=stage.at[h, pl.ds(k * s2, s2), :],
                send_sem=send_sems.at[dirn, h, k],
                recv_sem=recv_sems.at[dirn, h, k],
                device_id=(peer(dirn),),
                device_id_type=pl.DeviceIdType.MESH,
            )

        def ag_desc(dirn, h, k):
            region = sub(out_ref, ag_chunk(dirn, h), k)
            return pltpu.make_async_remote_copy(
                src_ref=region,
                dst_ref=region,
                send_sem=send_sems.at[dirn, h, k],
                recv_sem=ag_recv.at[dirn, h, k],
                device_id=(peer(dirn),),
                device_id_type=pl.DeviceIdType.MESH,
            )

        def rs_add(dirn, h, k):
            stage = stage_cw if dirn == CW else stage_ccw
            c8 = rs_acc_chunk(dirn, h)
            out_ref[pl.ds(c8 * s + k * s2, s2), :] += (
                stage[h, pl.ds(k * s2, s2), :]
            )

        issue_stage(0)

        barrier_sem = pltpu.get_barrier_semaphore()
        for nbr in (left, right):
            pl.semaphore_signal(
                barrier_sem, inc=1,
                device_id=(nbr,), device_id_type=pl.DeviceIdType.MESH,
            )

        cr0, cl0 = stage_chunks(0)
        ag_descs = {}
        for k in range(N_SUB):
            drain_group(cr0 * N_SUB + k, gsem.at[N_DEV - 1 + k])
            drain_group(cl0 * N_SUB + k, gsem.at[N_DEV - 1 + k])
            convert(cr0 * s + k * s2, s2)
            convert(cl0 * s + k * s2, s2)
            if k == 0:
                pl.semaphore_wait(barrier_sem, 2)
            rs_desc(CW, 0, k).start()
            rs_desc(CCW, 0, k).start()

        for h in range(N_DEV - 1):
            issue_stage(h + 1)
            finish_stage(h + 1)
            for k in range(N_SUB):
                for dirn in (CW, CCW):
                    rs_desc(dirn, h, k).wait()
                    rs_add(dirn, h, k)
                    if h + 1 < N_DEV - 1:
                        rs_desc(dirn, h + 1, k).start()
                    else:
                        ag = ag_desc(dirn, 0, k)
                        ag_descs[dirn, 0, k] = ag
                        ag.start()

        for h in range(1, N_DEV - 1):
            for k in range(N_SUB):
                for dirn in (CW, CCW):
                    ag_descs[dirn, h - 1, k].wait_recv()
                    ag = ag_desc(dirn, h, k)
                    ag_descs[dirn, h, k] = ag
                    ag.start()
        for k in range(N_SUB):
            for dirn in (CW, CCW):
                ag_descs[dirn, N_DEV - 2, k].wait_recv()
        for ag in ag_descs.values():
            ag.wait_send()

    return pl.pallas_call(
        body,
        out_shape=jax.ShapeDtypeStruct((n, d), jnp.bfloat16),
        in_specs=[
            pl.BlockSpec(memory_space=pltpu.SMEM),
            pl.BlockSpec(memory_space=pltpu.SMEM),
            pl.BlockSpec(memory_space=pltpu.SMEM),
            pl.BlockSpec(memory_space=pltpu.VMEM),
            pl.BlockSpec(memory_space=pltpu.HBM),
        ],
        out_specs=pl.BlockSpec(memory_space=pltpu.VMEM),
        scratch_shapes=[
            pltpu.VMEM((n, d), jnp.float32),
            pltpu.VMEM((N_DEV - 1, s, d), jnp.bfloat16),
            pltpu.VMEM((N_DEV - 1, s, d), jnp.bfloat16),
            pltpu.SemaphoreType.DMA((2, N_DEV - 1, N_SUB)),
            pltpu.SemaphoreType.DMA((2, N_DEV - 1, N_SUB)),
            pltpu.SemaphoreType.DMA((2, N_DEV - 1, N_SUB)),
            pltpu.SemaphoreType.DMA((N_DEV - 1 + N_SUB,)),
        ],
        compiler_params=pltpu.CompilerParams(collective_id=0),
    )(lidx, packed, ccounts, mask, table)


def kernel(table, idx):
    v_per, _ = table.shape
    n = idx.shape[0]
    my = lax.axis_index("i")
    lidx = idx.astype(jnp.int32) - my * v_per
    owned = ((lidx >= 0) & (lidx < v_per)).astype(jnp.int32)
    mask = owned.astype(jnp.float32)[:, None]
    n_groups = 2 * N_DEV * N_SUB
    s2 = n // n_groups
    owned2d = owned.reshape(n_groups, s2)
    ccounts = owned2d.sum(axis=1, dtype=jnp.int32)
    order = jnp.argsort(1 - owned2d, axis=1, stable=True)
    packed = (
        jnp.arange(n_groups, dtype=jnp.int32)[:, None] * s2
        + order.astype(jnp.int32)
    ).reshape(-1)
    return _fused(jnp.clip(lidx, 0, v_per - 1), packed, ccounts, mask, table)
